# baseline (speedup 1.0000x reference)
import jax
import jax.numpy as jnp
from jax import lax
from jax.experimental import pallas as pl
from jax.experimental.pallas import tpu as pltpu, tpu_sc as plsc

B, L, D = 4096, 201, 128
PL = 208
NC, NS, LANES = 2, 16, 16
NW = NC * NS
BPW = B // NW
NBUF = 4
HA = 104
HB = L - HA
NSTEP = 2 * BPW
OUTER = NSTEP // NBUF
REP = 2048
GPB = PL // LANES


def _sc_body(ids_hbm, len_hbm, table_hbm, out_hbm, clen_hbm,
             ids_v, rows_v, len_v, clen_v, semg, sems):
    wid = lax.axis_index("s") * NC + lax.axis_index("c")
    b0 = wid * BPW
    iota = lax.iota(jnp.int32, LANES)

    pltpu.sync_copy(len_hbm.at[pl.ds(b0, BPW)], len_v.at[pl.ds(0, BPW)])
    for j in range(BPW // LANES):
        sl = pl.ds(j * LANES, LANES)
        clen_v[sl] = len_v[sl] + 1
    pltpu.sync_copy(clen_v, clen_hbm.at[pl.ds(b0, BPW)])

    pltpu.sync_copy(ids_hbm.at[pl.ds(b0 * PL, BPW * PL)], ids_v)

    def remap_batch(k, carry):
        lnk = len_v[pl.ds(k, LANES)][0]
        base = k * PL
        for g in range(GPB):
            sl = pl.ds(base + g * LANES, LANES)
            pos = iota + g * LANES
            phase = jnp.bitwise_and(pos + k * 31, REP - 1)
            ids = ids_v[sl]
            ids = jnp.where(pos == lnk, 3, ids)
            ids_v[sl] = ids + phase * 4
        return carry

    lax.fori_loop(0, BPW, remap_batch, 0)

    def g_start(c, bb):
        p = bb % 2
        n = HB if p else HA
        off = (c >> 1) * PL + (HA if p else 0)
        pltpu.async_copy(table_hbm.at[ids_v.at[pl.ds(off, n)]],
                         rows_v.at[bb, pl.ds(0, n)], semg.at[bb])

    def g_wait(bb):
        p = bb % 2
        n = HB if p else HA
        pltpu.make_async_copy(table_hbm.at[ids_v.at[pl.ds(0, n)]],
                              rows_v.at[bb, pl.ds(0, n)], semg.at[bb]).wait()

    def s_start(c, bb):
        p = bb % 2
        n = HB if p else HA
        l0 = HA if p else 0
        pltpu.async_copy(rows_v.at[bb, pl.ds(0, n)],
                         out_hbm.at[b0 + (c >> 1), pl.ds(l0, n)],
                         sems.at[bb])

    def s_wait(bb):
        p = bb % 2
        n = HB if p else HA
        l0 = HA if p else 0
        pltpu.make_async_copy(rows_v.at[bb, pl.ds(0, n)],
                              out_hbm.at[0, pl.ds(l0, n)], sems.at[bb]).wait()

    g_start(0, 0)
    g_start(1, 1)

    def outer(o, carry):
        for bb in range(NBUF):
            c = o * NBUF + bb
            g_wait(bb)
            s_start(c, bb)
            nxt = c + 2
            bn = (bb + 2) % NBUF

            @pl.when(jnp.logical_and(c >= 2, nxt < NSTEP))
            def _():
                s_wait(bn)

            @pl.when(nxt < NSTEP)
            def _():
                g_start(nxt, bn)
        return carry

    lax.fori_loop(0, OUTER, outer, 0)
    for bb in range(NBUF):
        s_wait(bb)


def kernel(word_ids, lengths, table, eof_embedding):
    table4 = jnp.concatenate([table, eof_embedding], axis=0)
    table_rep = jnp.tile(table4, (REP, 1))
    ids_flat = jnp.pad(word_ids, ((0, 0), (0, PL - L))).reshape(B * PL)

    mesh = plsc.VectorSubcoreMesh(core_axis_name="c", subcore_axis_name="s")
    rep, char_len = pl.kernel(
        _sc_body,
        out_type=(
            jax.ShapeDtypeStruct((B, L, D), jnp.float32),
            jax.ShapeDtypeStruct((B,), jnp.int32),
        ),
        mesh=mesh,
        scratch_types=[
            pltpu.VMEM((BPW * PL,), jnp.int32),
            pltpu.VMEM((NBUF, HA, D), jnp.float32),
            pltpu.VMEM((BPW + LANES,), jnp.int32),
            pltpu.VMEM((BPW,), jnp.int32),
            pltpu.SemaphoreType.DMA((NBUF,)),
            pltpu.SemaphoreType.DMA((NBUF,)),
        ],
    )(ids_flat, lengths, table_rep)

    return rep, char_len


# speedup vs baseline: 1.0989x; 1.0989x over previous
import jax
import jax.numpy as jnp
from jax import lax
from jax.experimental import pallas as pl
from jax.experimental.pallas import tpu as pltpu, tpu_sc as plsc

B, L, D = 4096, 201, 128
PL = 208
NC, NS, LANES = 2, 16, 16
NW = NC * NS
BPW = B // NW
RPW = BPW * PL
CHUNK = 128
NCHUNK = RPW // CHUNK
NBUF = 4
OUTER = NCHUNK // NBUF
LOOKAHEAD = 2
REP = 2048
GPC = CHUNK // LANES


def _sc_body(ids_hbm, len_hbm, table_hbm, out_hbm, clen_hbm,
             ids_v, rows_v, len_v, eof_idx_v, eof_fill_v, eof_rows_v,
             clen_v, semg, sems):
    wid = lax.axis_index("s") * NC + lax.axis_index("c")
    row0 = wid * RPW

    pltpu.sync_copy(ids_hbm.at[pl.ds(wid * NCHUNK, NCHUNK)], ids_v)

    iota = lax.iota(jnp.int32, LANES)

    def remap_chunk(c, carry):
        for g in range(GPC):
            sl = pl.ds(g * LANES, LANES)
            phase = jnp.bitwise_and(iota + (c * CHUNK + g * LANES), REP - 1)
            ids_v[c, sl] = ids_v[c, sl] + phase * 4
        return carry

    lax.fori_loop(0, NCHUNK, remap_chunk, 0)

    def g_start(c, b):
        pltpu.async_copy(table_hbm.at[ids_v.at[c]], rows_v.at[b], semg.at[b])

    def g_wait(b):
        pltpu.make_async_copy(table_hbm.at[ids_v.at[0]], rows_v.at[b],
                              semg.at[b]).wait()

    def s_start(c, b):
        pltpu.async_copy(rows_v.at[b],
                         out_hbm.at[pl.ds(row0 + c * CHUNK, CHUNK)],
                         sems.at[b])

    def s_wait(b):
        pltpu.make_async_copy(rows_v.at[b], out_hbm.at[pl.ds(0, CHUNK)],
                              sems.at[b]).wait()

    for b in range(LOOKAHEAD):
        g_start(b, b)

    def outer(o, carry):
        for b in range(NBUF):
            c = o * NBUF + b
            g_wait(b)
            s_start(c, b)
            nxt = c + LOOKAHEAD
            bn = (b + LOOKAHEAD) % NBUF

            @pl.when(jnp.logical_and(c >= LOOKAHEAD, nxt < NCHUNK))
            def _():
                s_wait(bn)

            @pl.when(nxt < NCHUNK)
            def _():
                g_start(nxt, bn)
        return carry

    lax.fori_loop(0, OUTER, outer, 0)
    for b in range(NBUF):
        s_wait(b)

    b0 = wid * BPW
    pltpu.sync_copy(len_hbm.at[pl.ds(b0, BPW)], len_v)
    for j in range(BPW // LANES):
        sl = pl.ds(j * LANES, LANES)
        ln = len_v[sl]
        bi = iota + (b0 + j * LANES)
        eof_idx_v[sl] = bi * PL + ln
        clen_v[sl] = ln + 1
        phase = jnp.bitwise_and(iota + j * LANES, REP - 1)
        eof_fill_v[sl] = phase * 4 + 3
    pltpu.sync_copy(clen_v, clen_hbm.at[pl.ds(b0, BPW)])
    pltpu.async_copy(table_hbm.at[eof_fill_v], eof_rows_v, semg.at[0]).wait()
    pltpu.async_copy(eof_rows_v, out_hbm.at[eof_idx_v], semg.at[0]).wait()


def kernel(word_ids, lengths, table, eof_embedding):
    table4 = jnp.concatenate([table, eof_embedding], axis=0)
    table_rep = jnp.tile(table4, (REP, 1))
    ids_pad = jnp.pad(word_ids, ((0, 0), (0, PL - L)))
    ids2d = ids_pad.reshape(B * PL // CHUNK, CHUNK)

    mesh = plsc.VectorSubcoreMesh(core_axis_name="c", subcore_axis_name="s")
    out2d, char_len = pl.kernel(
        _sc_body,
        out_type=(
            jax.ShapeDtypeStruct((B * PL, D), jnp.float32),
            jax.ShapeDtypeStruct((B,), jnp.int32),
        ),
        mesh=mesh,
        scratch_types=[
            pltpu.VMEM((NCHUNK, CHUNK), jnp.int32),
            pltpu.VMEM((NBUF, CHUNK, D), jnp.float32),
            pltpu.VMEM((BPW,), jnp.int32),
            pltpu.VMEM((BPW,), jnp.int32),
            pltpu.VMEM((BPW,), jnp.int32),
            pltpu.VMEM((BPW, D), jnp.float32),
            pltpu.VMEM((BPW,), jnp.int32),
            pltpu.SemaphoreType.DMA((NBUF,)),
            pltpu.SemaphoreType.DMA((NBUF,)),
        ],
    )(ids2d, lengths, table_rep)

    rep = out2d.reshape(B, PL, D)[:, :L, :]
    return rep, char_len
